# Initial kernel scaffold; baseline (speedup 1.0000x reference)
import jax
import jax.numpy as jnp
from jax.experimental import pallas as pl


def kernel(x, edge_index, edge_attr, W1, b1, W2, b2, W3, b3, W4, b4, prelu_a):
    raise NotImplementedError("write your pallas kernel here")


# speedup vs baseline: 6.7242x; 6.7242x over previous
import functools

import jax
import jax.numpy as jnp
from jax import lax
from jax.experimental import pallas as pl
from jax.experimental.pallas import tpu as pltpu
from jax.experimental.pallas import tpu_sc as plsc

NC = 2
NS = 16
NW = NC * NS
L = 16
CH = 128

_BCAST_DNUMS = lax.GatherDimensionNumbers(
    offset_dims=(), collapsed_slice_dims=(0,), start_index_map=(0,))


def _lane_bcast(vec, j):
  idx = jnp.full((L, 1), j, jnp.int32)
  return lax.gather(vec, idx, _BCAST_DNUMS, (1,),
                    mode=lax.GatherScatterMode.PROMISE_IN_BOUNDS)


def _deg_body(dstp, ewp, out, dst_v, ew_v, buf_v, acc_sh, *,
              chunks_per_w, rows_per_tile):
  c = lax.axis_index("c")
  s = lax.axis_index("s")
  w = s * NC + c

  def _z(i, _):
    for f in range(8):
      buf_v[i, pl.ds(f * L, L)] = jnp.zeros((L,), jnp.float32)
    return 0
  lax.fori_loop(0, CH, _z, 0)
  for r in range(rows_per_tile // CH):
    pltpu.sync_copy(buf_v, acc_sh.at[pl.ds(s * rows_per_tile + r * CH, CH), :])
  plsc.subcore_barrier()

  iota = lax.iota(jnp.int32, L)
  onehot0 = jnp.where(iota == 0, 1.0, 0.0).astype(jnp.float32)

  def _chunk(t, _):
    base = (w * chunks_per_w + t) * CH
    pltpu.sync_copy(dstp.at[pl.ds(base, CH)], dst_v)
    pltpu.sync_copy(ewp.at[pl.ds(base, CH)], ew_v)

    def _fill(g, _):
      vec = ew_v[pl.ds(pl.multiple_of(g * L, L), L)]
      for j in range(L):
        buf_v[g * L + j, pl.ds(0, L)] = _lane_bcast(vec, j) * onehot0
      return 0
    lax.fori_loop(0, CH // L, _fill, 0)
    pltpu.sync_copy(buf_v, acc_sh.at[dst_v], add=True)
    return 0
  lax.fori_loop(0, chunks_per_w, _chunk, 0)
  plsc.subcore_barrier()

  for r in range(rows_per_tile // CH):
    row0 = s * rows_per_tile + r * CH
    pltpu.sync_copy(acc_sh.at[pl.ds(row0, CH), :], buf_v)
    pltpu.sync_copy(buf_v, out.at[c, pl.ds(row0, CH), :])


def _agg_body(hp, srcp, dstp, ewp, out, src_v, dst_v, ew_v, rows_v,
              acc_sh, sem, *, chunks_per_w, rows_per_tile, d):
  c = lax.axis_index("c")
  s = lax.axis_index("s")
  w = s * NC + c
  nf = d // L

  def _z(i, _):
    for f in range(nf):
      rows_v[i, pl.ds(f * L, L)] = jnp.zeros((L,), jnp.float32)
    return 0
  lax.fori_loop(0, CH, _z, 0)
  for r in range(rows_per_tile // CH):
    pltpu.sync_copy(rows_v, acc_sh.at[pl.ds(s * rows_per_tile + r * CH, CH), :])
  plsc.subcore_barrier()

  def _chunk(t, _):
    base = (w * chunks_per_w + t) * CH
    pltpu.sync_copy(srcp.at[pl.ds(base, CH)], src_v)
    pltpu.sync_copy(ewp.at[pl.ds(base, CH)], ew_v)
    pltpu.async_copy(hp.at[src_v], rows_v, sem).wait()

    def _scale(g, _):
      vec = ew_v[pl.ds(pl.multiple_of(g * L, L), L)]
      for j in range(L):
        ewb = _lane_bcast(vec, j)
        e = g * L + j
        for f in range(nf):
          sl = pl.ds(f * L, L)
          rows_v[e, sl] = rows_v[e, sl] * ewb
      return 0
    lax.fori_loop(0, CH // L, _scale, 0)

    pltpu.sync_copy(dstp.at[pl.ds(base, CH)], dst_v)
    pltpu.sync_copy(rows_v, acc_sh.at[dst_v], add=True)
    return 0
  lax.fori_loop(0, chunks_per_w, _chunk, 0)
  plsc.subcore_barrier()

  for r in range(rows_per_tile // CH):
    row0 = s * rows_per_tile + r * CH
    pltpu.sync_copy(acc_sh.at[pl.ds(row0, CH), :], rows_v)
    pltpu.sync_copy(rows_v, out.at[c, pl.ds(row0, CH), :])


def _tc_first_body(x_ref, w_ref, deg_ref, hp_ref, dinv_ref, *, n):
  deg = 1.0 + deg_ref[0] + deg_ref[1]
  dinv_all = lax.rsqrt(deg)
  dinv = dinv_all[:n, 0:1]
  h = jnp.dot(x_ref[...], w_ref[...], preferred_element_type=jnp.float32)
  hp_ref[...] = h * dinv
  dinv_ref[...] = dinv


def _tc_mid_body(acc_ref, hp_ref, dinv_ref, b_ref, w_ref, out_ref, *, n):
  dinv = dinv_ref[...]
  g = dinv * (acc_ref[0, :n, :] + acc_ref[1, :n, :] + hp_ref[...]) + b_ref[...]
  out_ref[...] = dinv * jnp.dot(g, w_ref[...],
                                preferred_element_type=jnp.float32)


def _tc_last_body(acc_ref, hp_ref, dinv_ref, b_ref, a_ref, out_ref, *, n):
  g = dinv_ref[...] * (acc_ref[0, :n, :] + acc_ref[1, :n, :] + hp_ref[...]) \
      + b_ref[...]
  out_ref[...] = jnp.where(g > 0, g, a_ref[...] * g)


def kernel(x, edge_index, edge_attr, W1, b1, W2, b2, W3, b3, W4, b4, prelu_a):
  n, d_in = x.shape
  d = W1.shape[1]
  e = edge_index.shape[1]
  f32 = jnp.float32

  rows_per_tile = ((n + NS * CH - 1) // (NS * CH)) * CH
  n_pad = NS * rows_per_tile
  chunks_per_w = (e + NW * CH - 1) // (NW * CH)
  e_pad = NW * CH * chunks_per_w
  pad = e_pad - e

  src_p = jnp.concatenate([edge_index[0], jnp.zeros((pad,), jnp.int32)])
  dst_p = jnp.concatenate([edge_index[1], jnp.zeros((pad,), jnp.int32)])
  ew_p = jnp.concatenate([edge_attr, jnp.zeros((pad,), f32)])

  mesh = plsc.VectorSubcoreMesh(core_axis_name="c", subcore_axis_name="s")

  deg_fn = pl.kernel(
      functools.partial(_deg_body, chunks_per_w=chunks_per_w,
                        rows_per_tile=rows_per_tile),
      out_type=jax.ShapeDtypeStruct((NC, n_pad, 128), f32),
      mesh=mesh,
      scratch_types=[
          pltpu.VMEM((CH,), jnp.int32),
          pltpu.VMEM((CH,), f32),
          pltpu.VMEM((CH, 128), f32),
          pltpu.VMEM_SHARED((n_pad, 128), f32),
      ],
  )
  deg2 = deg_fn(dst_p, ew_p)

  agg_fn = pl.kernel(
      functools.partial(_agg_body, chunks_per_w=chunks_per_w,
                        rows_per_tile=rows_per_tile, d=d),
      out_type=jax.ShapeDtypeStruct((NC, n_pad, d), f32),
      mesh=mesh,
      scratch_types=[
          pltpu.VMEM((CH,), jnp.int32),
          pltpu.VMEM((CH,), jnp.int32),
          pltpu.VMEM((CH,), f32),
          pltpu.VMEM((CH, d), f32),
          pltpu.VMEM_SHARED((n_pad, d), f32),
          pltpu.SemaphoreType.DMA,
      ],
  )

  tc_first = pl.pallas_call(
      functools.partial(_tc_first_body, n=n),
      out_shape=(jax.ShapeDtypeStruct((n, d), f32),
                 jax.ShapeDtypeStruct((n, 1), f32)),
  )
  tc_mid = pl.pallas_call(
      functools.partial(_tc_mid_body, n=n),
      out_shape=jax.ShapeDtypeStruct((n, d), f32),
  )
  tc_last = pl.pallas_call(
      functools.partial(_tc_last_body, n=n),
      out_shape=jax.ShapeDtypeStruct((n, d), f32),
  )

  hp, dinv = tc_first(x, W1, deg2)
  acc = agg_fn(hp, src_p, dst_p, ew_p)
  hp = tc_mid(acc, hp, dinv, b1.reshape(1, d), W2)
  acc = agg_fn(hp, src_p, dst_p, ew_p)
  hp = tc_mid(acc, hp, dinv, b2.reshape(1, d), W3)
  acc = agg_fn(hp, src_p, dst_p, ew_p)
  hp = tc_mid(acc, hp, dinv, b3.reshape(1, d), W4)
  acc = agg_fn(hp, src_p, dst_p, ew_p)
  out = tc_last(acc, hp, dinv, b4.reshape(1, d), prelu_a.reshape(1, d))
  return out
